# baseline (speedup 1.0000x reference)
import functools

import jax
import jax.numpy as jnp
from jax import lax
from jax.experimental import pallas as pl
from jax.experimental.pallas import tpu as pltpu
from jax.experimental.pallas import tpu_sc as plsc

_B = 4096
_K = 4096
_NTILES = 32
_FPT = _K // _NTILES
_CH = 8
_NCH = _FPT // _CH
_LANE = 16


def _make_layer(binarize: bool):
    mesh = plsc.VectorSubcoreMesh(core_axis_name="c", subcore_axis_name="s")

    @functools.partial(
        pl.kernel,
        out_type=jax.ShapeDtypeStruct((_K, _B), jnp.float32),
        mesh=mesh,
        scratch_types=[
            pltpu.VMEM((_FPT,), jnp.int32),
            pltpu.VMEM((_FPT,), jnp.int32),
            pltpu.VMEM((16, _FPT), jnp.float32),
            pltpu.VMEM((4, _FPT), jnp.float32),
            pltpu.VMEM((_CH, _B), jnp.float32),
            pltpu.VMEM((_CH, _B), jnp.float32),
            pltpu.VMEM((_CH, _B), jnp.float32),
            pltpu.SemaphoreType.DMA,
            pltpu.SemaphoreType.DMA,
        ],
        name="logic_layer_bin" if binarize else "logic_layer_f32",
    )
    def layer(h_hbm, wt_hbm, ia_hbm, ib_hbm, out_hbm,
              ia_v, ib_v, wt_v, kc_v, ra_v, rb_v, o_v, sem_a, sem_b):
        wid = lax.axis_index("s") * 2 + lax.axis_index("c")
        base = wid * _FPT
        pltpu.sync_copy(ia_hbm.at[pl.ds(base, _FPT)], ia_v)
        pltpu.sync_copy(ib_hbm.at[pl.ds(base, _FPT)], ib_v)
        pltpu.sync_copy(wt_hbm.at[:, pl.ds(base, _FPT)], wt_v)

        for g in range(_FPT // _LANE):
            sl = pl.ds(g * _LANE, _LANE)
            e = [wt_v[i, sl] for i in range(16)]
            m = e[0]
            for i in range(1, 16):
                m = jnp.maximum(m, e[i])
            e = [jnp.exp(v - m) for v in e]
            s = e[0]
            for i in range(1, 16):
                s = s + e[i]
            inv = 1.0 / s
            k0 = e[8] + e[9] + e[10] + e[11] + e[12] + e[13] + e[14] + e[15]
            k1 = e[2] + e[3] + e[6] + e[7] - e[8] - e[9] - e[12] - e[13]
            k2 = e[4] + e[5] + e[6] + e[7] - e[8] - e[9] - e[10] - e[11]
            k3 = (e[1] - e[2] - e[4] - 2.0 * e[6] - e[7] + e[8]
                  + 2.0 * e[9] + e[11] + e[13] - e[14])
            kc_v[0, sl] = k0 * inv
            kc_v[1, sl] = k1 * inv
            kc_v[2, sl] = k2 * inv
            kc_v[3, sl] = k3 * inv

        for c in range(_NCH):
            cpa = pltpu.async_copy(h_hbm.at[ia_v.at[pl.ds(c * _CH, _CH)]],
                                   ra_v, sem_a)
            cpb = pltpu.async_copy(h_hbm.at[ib_v.at[pl.ds(c * _CH, _CH)]],
                                   rb_v, sem_b)
            cpa.wait()
            cpb.wait()
            for f in range(_CH):
                j = c * _CH + f
                grp = pl.ds((j // _LANE) * _LANE, _LANE)
                k0 = kc_v[0, grp][j % _LANE]
                k1 = kc_v[1, grp][j % _LANE]
                k2 = kc_v[2, grp][j % _LANE]
                k3 = kc_v[3, grp][j % _LANE]

                def bat_body(i, carry2, f=f, k0=k0, k1=k1, k2=k2, k3=k3):
                    off = i * _LANE
                    av = ra_v[f, pl.ds(off, _LANE)]
                    bv = rb_v[f, pl.ds(off, _LANE)]
                    if binarize:
                        av = jnp.where(av > 0.0, 1.0, 0.0)
                        bv = jnp.where(bv > 0.0, 1.0, 0.0)
                    o_v[f, pl.ds(off, _LANE)] = (
                        k0 + k1 * av + k2 * bv + k3 * (av * bv))
                    return carry2

                lax.fori_loop(0, _B // _LANE, bat_body, 0)
            pltpu.sync_copy(o_v, out_hbm.at[pl.ds(base + c * _CH, _CH)])

    return layer


_layer0 = _make_layer(binarize=True)
_layerK = _make_layer(binarize=False)


def _groupsum(h3):
    cb = 512

    def body(h_ref, o_ref):
        hb = h_ref[...]
        o_ref[0, :] = jnp.sum(hb[: _K // 2, :], axis=0)
        o_ref[1, :] = jnp.sum(hb[_K // 2:, :], axis=0)

    return pl.pallas_call(
        body,
        grid=(_B // cb,),
        in_specs=[pl.BlockSpec((_K, cb), lambda i: (0, i))],
        out_specs=pl.BlockSpec((2, cb), lambda i: (0, i)),
        out_shape=jax.ShapeDtypeStruct((2, _B), jnp.float32),
    )(h3)


def kernel(x, w0, w1, w2, w3, a0, b0, a1, b1, a2, b2, a3, b3):
    x_t = jnp.transpose(x)
    h = _layer0(x_t, jnp.transpose(w0), a0, b0)
    h = _layerK(h, jnp.transpose(w1), a1, b1)
    h = _layerK(h, jnp.transpose(w2), a2, b2)
    h = _layerK(h, jnp.transpose(w3), a3, b3)
    out = _groupsum(h)
    return jnp.transpose(out)


# speedup vs baseline: 2.7487x; 2.7487x over previous
import functools

import jax
import jax.numpy as jnp
from jax import lax
from jax.experimental import pallas as pl
from jax.experimental.pallas import tpu as pltpu
from jax.experimental.pallas import tpu_sc as plsc

_B = 4096
_BH = _B // 2
_K = 4096
_NTILES = 32
_FPT = _K // _NTILES
_CH = 8
_NCH = _FPT // _CH
_LANE = 16
_UNR = 8

_mesh = plsc.VectorSubcoreMesh(core_axis_name="c", subcore_axis_name="s")
_half_out = (jax.ShapeDtypeStruct((_K, _BH), jnp.float32),
             jax.ShapeDtypeStruct((_K, _BH), jnp.float32))


def _fold_coeffs(wt_v, kc_v):
    for g in range(_FPT // _LANE):
        sl = pl.ds(g * _LANE, _LANE)
        e = [wt_v[i, sl] for i in range(16)]
        m = e[0]
        for i in range(1, 16):
            m = jnp.maximum(m, e[i])
        e = [jnp.exp(v - m) for v in e]
        s = e[0]
        for i in range(1, 16):
            s = s + e[i]
        inv = 1.0 / s
        k0 = e[8] + e[9] + e[10] + e[11] + e[12] + e[13] + e[14] + e[15]
        k1 = e[2] + e[3] + e[6] + e[7] - e[8] - e[9] - e[12] - e[13]
        k2 = e[4] + e[5] + e[6] + e[7] - e[8] - e[9] - e[10] - e[11]
        k3 = (e[1] - e[2] - e[4] - 2.0 * e[6] - e[7] + e[8]
              + 2.0 * e[9] + e[11] + e[13] - e[14])
        kc_v[pl.ds(0 * _FPT + g * _LANE, _LANE)] = k0 * inv
        kc_v[pl.ds(1 * _FPT + g * _LANE, _LANE)] = k1 * inv
        kc_v[pl.ds(2 * _FPT + g * _LANE, _LANE)] = k2 * inv
        kc_v[pl.ds(3 * _FPT + g * _LANE, _LANE)] = k3 * inv


@functools.partial(
    pl.kernel,
    out_type=_half_out,
    mesh=_mesh,
    scratch_types=[
        pltpu.VMEM((_FPT,), jnp.int32),
        pltpu.VMEM((_FPT,), jnp.int32),
        pltpu.VMEM((16, _FPT), jnp.float32),
        pltpu.VMEM((4 * _FPT,), jnp.float32),
        pltpu.VMEM((_CH, _BH), jnp.float32),
        pltpu.VMEM((_CH, _BH), jnp.float32),
        pltpu.VMEM((_CH, _BH), jnp.float32),
        pltpu.VMEM((_CH, _BH), jnp.float32),
        pltpu.VMEM((_CH, _BH), jnp.float32),
        pltpu.VMEM((_CH, _BH), jnp.float32),
        pltpu.SemaphoreType.DMA,
        pltpu.SemaphoreType.DMA,
        pltpu.SemaphoreType.DMA,
        pltpu.SemaphoreType.DMA,
    ],
    name="logic_layer_mid",
)
def _layer_mid(hL, hR, wt_hbm, ia_hbm, ib_hbm, oL_hbm, oR_hbm,
               ia_v, ib_v, wt_v, kc_v, ra0, rb0, ra1, rb1, o0, o1,
               sa0, sb0, sa1, sb1):
    wid = lax.axis_index("s") * 2 + lax.axis_index("c")
    base = wid * _FPT
    pltpu.sync_copy(ia_hbm.at[pl.ds(base, _FPT)], ia_v)
    pltpu.sync_copy(ib_hbm.at[pl.ds(base, _FPT)], ib_v)
    pltpu.sync_copy(wt_hbm.at[:, pl.ds(base, _FPT)], wt_v)
    _fold_coeffs(wt_v, kc_v)

    def compute_chunk(kv, lane_off, ra, rb, o):
        kv0, kv1, kv2, kv3 = kv
        for f in range(_CH):
            k0 = kv0[lane_off + f]
            k1 = kv1[lane_off + f]
            k2 = kv2[lane_off + f]
            k3 = kv3[lane_off + f]

            @plsc.parallel_loop(0, _BH, step=_LANE, unroll=_UNR)
            def bat(off, f=f, k0=k0, k1=k1, k2=k2, k3=k3):
                av = ra[f, pl.ds(off, _LANE)]
                bv = rb[f, pl.ds(off, _LANE)]
                o[f, pl.ds(off, _LANE)] = (
                    (k1 + k3 * bv) * av + (k2 * bv + k0))

    def gath(tab, idx_v, c, dst, sem):
        return pltpu.async_copy(tab.at[idx_v.at[pl.ds(c * _CH, _CH)]],
                                dst, sem)

    @pl.loop(0, _NCH // 2)
    def body(t):
        c0 = 2 * t
        c1 = 2 * t + 1
        kv = [kc_v[pl.ds(n * _FPT + t * _LANE, _LANE)] for n in range(4)]
        osl0 = pl.ds(base + c0 * _CH, _CH)
        osl1 = pl.ds(base + c1 * _CH, _CH)

        cpa = gath(hL, ia_v, c0, ra0, sa0)
        cpb = gath(hL, ib_v, c0, rb0, sb0)
        cpa1 = gath(hR, ia_v, c0, ra1, sa1)
        cpb1 = gath(hR, ib_v, c0, rb1, sb1)
        cpa.wait()
        cpb.wait()
        compute_chunk(kv, 0, ra0, rb0, o0)
        cpa = gath(hL, ia_v, c1, ra0, sa0)
        cpb = gath(hL, ib_v, c1, rb0, sb0)
        pltpu.sync_copy(o0, oL_hbm.at[osl0])
        cpa1.wait()
        cpb1.wait()
        compute_chunk(kv, 0, ra1, rb1, o1)
        cpa1 = gath(hR, ia_v, c1, ra1, sa1)
        cpb1 = gath(hR, ib_v, c1, rb1, sb1)
        pltpu.sync_copy(o1, oR_hbm.at[osl0])
        cpa.wait()
        cpb.wait()
        compute_chunk(kv, _CH, ra0, rb0, o0)
        pltpu.sync_copy(o0, oL_hbm.at[osl1])
        cpa1.wait()
        cpb1.wait()
        compute_chunk(kv, _CH, ra1, rb1, o1)
        pltpu.sync_copy(o1, oR_hbm.at[osl1])


@functools.partial(
    pl.kernel,
    out_type=_half_out,
    mesh=_mesh,
    scratch_types=[
        pltpu.VMEM((_FPT,), jnp.int32),
        pltpu.VMEM((_FPT,), jnp.int32),
        pltpu.VMEM((16, _FPT), jnp.float32),
        pltpu.VMEM((4 * _FPT,), jnp.float32),
        pltpu.VMEM((2 * _B,), jnp.float32),
        pltpu.VMEM((_LANE, _BH), jnp.float32),
        pltpu.VMEM((_LANE, _BH), jnp.float32),
    ],
    name="logic_layer_in",
)
def _layer_in(x_hbm, wt_hbm, ia_hbm, ib_hbm, oL_hbm, oR_hbm,
              ia_v, ib_v, wt_v, kc_v, tx_v, oL_v, oR_v):
    wid = lax.axis_index("s") * 2 + lax.axis_index("c")
    base = wid * _FPT
    pltpu.sync_copy(ia_hbm.at[pl.ds(base, _FPT)], ia_v)
    pltpu.sync_copy(ib_hbm.at[pl.ds(base, _FPT)], ib_v)
    pltpu.sync_copy(wt_hbm.at[:, pl.ds(base, _FPT)], wt_v)
    pltpu.sync_copy(x_hbm, tx_v)
    _fold_coeffs(wt_v, kc_v)

    @plsc.parallel_loop(0, 2 * _B, step=_LANE, unroll=_UNR)
    def binz(off):
        sl = pl.ds(off, _LANE)
        tx_v[sl] = jnp.where(tx_v[sl] > 0.0, 1.0, 0.0)

    @pl.loop(0, _FPT // _LANE)
    def grp_body(g):
        iag = ia_v[pl.ds(g * _LANE, _LANE)]
        ibg = ib_v[pl.ds(g * _LANE, _LANE)]
        kv0 = kc_v[pl.ds(0 * _FPT + g * _LANE, _LANE)]
        kv1 = kc_v[pl.ds(1 * _FPT + g * _LANE, _LANE)]
        kv2 = kc_v[pl.ds(2 * _FPT + g * _LANE, _LANE)]
        kv3 = kc_v[pl.ds(3 * _FPT + g * _LANE, _LANE)]

        for f in range(_LANE):
            abase = iag[f] * _B
            bbase = ibg[f] * _B
            k0 = kv0[f]
            k1 = kv1[f]
            k2 = kv2[f]
            k3 = kv3[f]

            @plsc.parallel_loop(0, _BH, step=_LANE, unroll=_UNR // 2)
            def bat(off, f=f, abase=abase, bbase=bbase,
                    k0=k0, k1=k1, k2=k2, k3=k3):
                avL = tx_v[pl.ds(abase + off, _LANE)]
                bvL = tx_v[pl.ds(bbase + off, _LANE)]
                oL_v[f, pl.ds(off, _LANE)] = (
                    (k1 + k3 * bvL) * avL + (k2 * bvL + k0))
                avR = tx_v[pl.ds(abase + _BH + off, _LANE)]
                bvR = tx_v[pl.ds(bbase + _BH + off, _LANE)]
                oR_v[f, pl.ds(off, _LANE)] = (
                    (k1 + k3 * bvR) * avR + (k2 * bvR + k0))

        osl = pl.ds(base + g * _LANE, _LANE)
        pltpu.sync_copy(oL_v, oL_hbm.at[osl])
        pltpu.sync_copy(oR_v, oR_hbm.at[osl])


_part_out = (jax.ShapeDtypeStruct((_NTILES, _BH), jnp.float32),
             jax.ShapeDtypeStruct((_NTILES, _BH), jnp.float32))


@functools.partial(
    pl.kernel,
    out_type=_part_out,
    mesh=_mesh,
    scratch_types=[
        pltpu.VMEM((_FPT,), jnp.int32),
        pltpu.VMEM((_FPT,), jnp.int32),
        pltpu.VMEM((16, _FPT), jnp.float32),
        pltpu.VMEM((4 * _FPT,), jnp.float32),
        pltpu.VMEM((_CH, _BH), jnp.float32),
        pltpu.VMEM((_CH, _BH), jnp.float32),
        pltpu.VMEM((_CH, _BH), jnp.float32),
        pltpu.VMEM((_CH, _BH), jnp.float32),
        pltpu.VMEM((1, _BH), jnp.float32),
        pltpu.VMEM((1, _BH), jnp.float32),
        pltpu.SemaphoreType.DMA,
        pltpu.SemaphoreType.DMA,
        pltpu.SemaphoreType.DMA,
        pltpu.SemaphoreType.DMA,
    ],
    name="logic_layer_out",
)
def _layer_last(hL, hR, wt_hbm, ia_hbm, ib_hbm, pL_hbm, pR_hbm,
                ia_v, ib_v, wt_v, kc_v, ra0, rb0, ra1, rb1, accL, accR,
                sa0, sb0, sa1, sb1):
    wid = lax.axis_index("s") * 2 + lax.axis_index("c")
    base = wid * _FPT
    pltpu.sync_copy(ia_hbm.at[pl.ds(base, _FPT)], ia_v)
    pltpu.sync_copy(ib_hbm.at[pl.ds(base, _FPT)], ib_v)
    pltpu.sync_copy(wt_hbm.at[:, pl.ds(base, _FPT)], wt_v)
    _fold_coeffs(wt_v, kc_v)

    zv = wt_v[0, pl.ds(0, _LANE)] * 0.0

    @pl.loop(0, _BH // _LANE)
    def zinit(i):
        accL[0, pl.ds(i * _LANE, _LANE)] = zv
        accR[0, pl.ds(i * _LANE, _LANE)] = zv

    def accum_chunk(kv, lane_off, ra, rb, acc):
        kv0, kv1, kv2, kv3 = kv
        ks = [(kv0[lane_off + f], kv1[lane_off + f],
               kv2[lane_off + f], kv3[lane_off + f]) for f in range(_CH)]

        @plsc.parallel_loop(0, _BH, step=_LANE, unroll=4)
        def bat(off):
            sl = pl.ds(off, _LANE)
            accv = acc[0, sl]
            for f in range(_CH):
                k0, k1, k2, k3 = ks[f]
                av = ra[f, sl]
                bv = rb[f, sl]
                accv = accv + ((k1 + k3 * bv) * av + (k2 * bv + k0))
            acc[0, sl] = accv

    def gath(tab, idx_v, c, dst, sem):
        return pltpu.async_copy(tab.at[idx_v.at[pl.ds(c * _CH, _CH)]],
                                dst, sem)

    @pl.loop(0, _NCH // 2)
    def body(t):
        c0 = 2 * t
        c1 = 2 * t + 1
        kv = [kc_v[pl.ds(n * _FPT + t * _LANE, _LANE)] for n in range(4)]

        cpa = gath(hL, ia_v, c0, ra0, sa0)
        cpb = gath(hL, ib_v, c0, rb0, sb0)
        cpa1 = gath(hR, ia_v, c0, ra1, sa1)
        cpb1 = gath(hR, ib_v, c0, rb1, sb1)
        cpa.wait()
        cpb.wait()
        accum_chunk(kv, 0, ra0, rb0, accL)
        cpa = gath(hL, ia_v, c1, ra0, sa0)
        cpb = gath(hL, ib_v, c1, rb0, sb0)
        cpa1.wait()
        cpb1.wait()
        accum_chunk(kv, 0, ra1, rb1, accR)
        cpa1 = gath(hR, ia_v, c1, ra1, sa1)
        cpb1 = gath(hR, ib_v, c1, rb1, sb1)
        cpa.wait()
        cpb.wait()
        accum_chunk(kv, _CH, ra0, rb0, accL)
        cpa1.wait()
        cpb1.wait()
        accum_chunk(kv, _CH, ra1, rb1, accR)

    pltpu.sync_copy(accL, pL_hbm.at[pl.ds(wid, 1)])
    pltpu.sync_copy(accR, pR_hbm.at[pl.ds(wid, 1)])


def _combine(pL, pR):
    def body(l_ref, r_ref, o_ref):
        lv = l_ref[...]
        rv = r_ref[...]
        o_ref[0, : _BH] = jnp.sum(lv[:16, :], axis=0)
        o_ref[0, _BH:] = jnp.sum(rv[:16, :], axis=0)
        o_ref[1, : _BH] = jnp.sum(lv[16:, :], axis=0)
        o_ref[1, _BH:] = jnp.sum(rv[16:, :], axis=0)

    return pl.pallas_call(
        body,
        out_shape=jax.ShapeDtypeStruct((2, _B), jnp.float32),
    )(pL, pR)


def kernel(x, w0, w1, w2, w3, a0, b0, a1, b1, a2, b2, a3, b3):
    x_flat = jnp.transpose(x).reshape(-1)
    hL, hR = _layer_in(x_flat, jnp.transpose(w0), a0, b0)
    hL, hR = _layer_mid(hL, hR, jnp.transpose(w1), a1, b1)
    hL, hR = _layer_mid(hL, hR, jnp.transpose(w2), a2, b2)
    pL, pR = _layer_last(hL, hR, jnp.transpose(w3), a3, b3)
    return jnp.transpose(_combine(pL, pR))


# speedup vs baseline: 3.3518x; 1.2194x over previous
import functools

import jax
import jax.numpy as jnp
from jax import lax
from jax.experimental import pallas as pl
from jax.experimental.pallas import tpu as pltpu
from jax.experimental.pallas import tpu_sc as plsc

_B = 4096
_NQ = 4
_BQ = _B // _NQ
_K = 4096
_NTILES = 32
_FPT = _K // _NTILES
_CH = 8
_NCH = _FPT // _CH
_LANE = 16
_UNR = 8

_mesh = plsc.VectorSubcoreMesh(core_axis_name="c", subcore_axis_name="s")
_tab_out = jax.ShapeDtypeStruct((_NQ * _K, _BQ), jnp.float32)


def _fold_coeffs(wt_v, kc_v):
    for g in range(_FPT // _LANE):
        sl = pl.ds(g * _LANE, _LANE)
        e = [wt_v[i, sl] for i in range(16)]
        m = e[0]
        for i in range(1, 16):
            m = jnp.maximum(m, e[i])
        e = [jnp.exp(v - m) for v in e]
        s = e[0]
        for i in range(1, 16):
            s = s + e[i]
        inv = 1.0 / s
        k0 = e[8] + e[9] + e[10] + e[11] + e[12] + e[13] + e[14] + e[15]
        k1 = e[2] + e[3] + e[6] + e[7] - e[8] - e[9] - e[12] - e[13]
        k2 = e[4] + e[5] + e[6] + e[7] - e[8] - e[9] - e[10] - e[11]
        k3 = (e[1] - e[2] - e[4] - 2.0 * e[6] - e[7] + e[8]
              + 2.0 * e[9] + e[11] + e[13] - e[14])
        kc_v[pl.ds(0 * _FPT + g * _LANE, _LANE)] = k0 * inv
        kc_v[pl.ds(1 * _FPT + g * _LANE, _LANE)] = k1 * inv
        kc_v[pl.ds(2 * _FPT + g * _LANE, _LANE)] = k2 * inv
        kc_v[pl.ds(3 * _FPT + g * _LANE, _LANE)] = k3 * inv


def _load_quarter_indices(iab_hbm, iq_v, base):
    pltpu.sync_copy(iab_hbm.at[pl.ds(base * 2, 2 * _FPT)], iq_v.at[0])
    for q in range(1, _NQ):
        @plsc.parallel_loop(0, 2 * _FPT, step=_LANE)
        def shift(i, q=q):
            sl = pl.ds(i, _LANE)
            iq_v[q, sl] = iq_v[0, sl] + (q * _K)


@functools.partial(
    pl.kernel,
    out_type=_tab_out,
    mesh=_mesh,
    scratch_types=[
        pltpu.VMEM((_NQ, 2 * _FPT), jnp.int32),
        pltpu.VMEM((16, _FPT), jnp.float32),
        pltpu.VMEM((4 * _FPT,), jnp.float32),
        pltpu.VMEM((2 * _CH, _BQ), jnp.float32),
        pltpu.VMEM((2 * _CH, _BQ), jnp.float32),
        pltpu.VMEM((2 * _CH, _BQ), jnp.float32),
        pltpu.VMEM((2 * _CH, _BQ), jnp.float32),
        pltpu.VMEM((_CH, _BQ), jnp.float32),
        pltpu.VMEM((_CH, _BQ), jnp.float32),
        pltpu.SemaphoreType.DMA,
        pltpu.SemaphoreType.DMA,
        pltpu.SemaphoreType.DMA,
        pltpu.SemaphoreType.DMA,
        pltpu.SemaphoreType.DMA,
        pltpu.SemaphoreType.DMA,
    ],
    name="logic_layer_mid",
)
def _layer_mid(ht, wt_hbm, iab_hbm, ot_hbm,
               iq_v, wt_v, kc_v, rab0, rab1, rab2, rab3, o0, o1,
               sg0, sg1, sg2, sg3, so0, so1):
    wid = lax.axis_index("s") * 2 + lax.axis_index("c")
    base = wid * _FPT
    _load_quarter_indices(iab_hbm, iq_v, base)
    pltpu.sync_copy(wt_hbm.at[:, pl.ds(base, _FPT)], wt_v)
    _fold_coeffs(wt_v, kc_v)

    rabs = [rab0, rab1, rab2, rab3]
    sgs = [sg0, sg1, sg2, sg3]
    os_ = [o0, o1]
    sos = [so0, so1]

    def compute_chunk(kv, lane_off, rab, o):
        kv0, kv1, kv2, kv3 = kv
        for f in range(_CH):
            k0 = kv0[lane_off + f]
            k1 = kv1[lane_off + f]
            k2 = kv2[lane_off + f]
            k3 = kv3[lane_off + f]

            @plsc.parallel_loop(0, _BQ, step=_LANE, unroll=_UNR)
            def bat(off, f=f, k0=k0, k1=k1, k2=k2, k3=k3):
                av = rab[f, pl.ds(off, _LANE)]
                bv = rab[_CH + f, pl.ds(off, _LANE)]
                o[f, pl.ds(off, _LANE)] = (
                    (k1 + k3 * bv) * av + (k2 * bv + k0))

    def gath(j_slot, c, q):
        idx = iq_v.at[q, pl.ds(c * 2 * _CH, 2 * _CH)]
        return pltpu.make_async_copy(ht.at[idx], rabs[j_slot], sgs[j_slot])

    def owrite(o_slot, c, q):
        rows = pl.ds(q * _K + base + c * _CH, _CH)
        return pltpu.make_async_copy(os_[o_slot], ot_hbm.at[rows],
                                     sos[o_slot])

    for q in range(3):
        gath(q, 0, q).start()

    @pl.loop(0, _NCH // 2)
    def body(t):
        c0 = 2 * t
        c1 = 2 * t + 1
        kv = [kc_v[pl.ds(n * _FPT + t * _LANE, _LANE)] for n in range(4)]

        for u in range(8):
            c = c0 if u < 4 else c1
            lane_off = 0 if u < 4 else _CH
            q = u % 4
            slot = u % 4
            o_slot = u % 2

            if u + 3 < 8:
                gath((u + 3) % 4, c0 if u + 3 < 4 else c1, (u + 3) % 4
                     ).start()
            else:
                @pl.when(t < _NCH // 2 - 1)
                def _(u=u):
                    gath((u + 3) % 4, c0 + 2, (u + 3) % 4).start()

            gath(slot, c, q).wait()

            if u >= 2:
                owrite(o_slot, c, q).wait()
            else:
                @pl.when(t > 0)
                def _(o_slot=o_slot, c=c, q=q):
                    owrite(o_slot, c, q).wait()

            compute_chunk(kv, lane_off, rabs[slot], os_[o_slot])
            owrite(o_slot, c, q).start()

    owrite(0, _NCH - 1, 2).wait()
    owrite(1, _NCH - 1, 3).wait()


@functools.partial(
    pl.kernel,
    out_type=_tab_out,
    mesh=_mesh,
    scratch_types=[
        pltpu.VMEM((_FPT,), jnp.int32),
        pltpu.VMEM((_FPT,), jnp.int32),
        pltpu.VMEM((16, _FPT), jnp.float32),
        pltpu.VMEM((4 * _FPT,), jnp.float32),
        pltpu.VMEM((2 * _B,), jnp.float32),
        pltpu.VMEM((_LANE, _BQ), jnp.float32),
        pltpu.VMEM((_LANE, _BQ), jnp.float32),
        pltpu.SemaphoreType.DMA,
        pltpu.SemaphoreType.DMA,
    ],
    name="logic_layer_in",
)
def _layer_in(x_hbm, wt_hbm, ia_hbm, ib_hbm, ot_hbm,
              ia_v, ib_v, wt_v, kc_v, tx_v, o0, o1, so0, so1):
    wid = lax.axis_index("s") * 2 + lax.axis_index("c")
    base = wid * _FPT
    pltpu.sync_copy(ia_hbm.at[pl.ds(base, _FPT)], ia_v)
    pltpu.sync_copy(ib_hbm.at[pl.ds(base, _FPT)], ib_v)
    pltpu.sync_copy(wt_hbm.at[:, pl.ds(base, _FPT)], wt_v)
    pltpu.sync_copy(x_hbm, tx_v)
    _fold_coeffs(wt_v, kc_v)

    os_ = [o0, o1]
    sos = [so0, so1]

    @plsc.parallel_loop(0, 2 * _B, step=_LANE, unroll=_UNR)
    def binz(off):
        sl = pl.ds(off, _LANE)
        tx_v[sl] = jnp.where(tx_v[sl] > 0.0, 1.0, 0.0)

    @pl.loop(0, _FPT // _LANE)
    def grp_body(g):
        iag = ia_v[pl.ds(g * _LANE, _LANE)]
        ibg = ib_v[pl.ds(g * _LANE, _LANE)]
        kv0 = kc_v[pl.ds(0 * _FPT + g * _LANE, _LANE)]
        kv1 = kc_v[pl.ds(1 * _FPT + g * _LANE, _LANE)]
        kv2 = kc_v[pl.ds(2 * _FPT + g * _LANE, _LANE)]
        kv3 = kc_v[pl.ds(3 * _FPT + g * _LANE, _LANE)]

        for q in range(_NQ):
            o_slot = q % 2
            rows = pl.ds(q * _K + base + g * _LANE, _LANE)

            if q >= 2:
                pltpu.make_async_copy(os_[o_slot], ot_hbm.at[rows],
                                      sos[o_slot]).wait()
            else:
                @pl.when(g > 0)
                def _(o_slot=o_slot, rows=rows):
                    pltpu.make_async_copy(os_[o_slot], ot_hbm.at[rows],
                                          sos[o_slot]).wait()

            for f in range(_LANE):
                abase = iag[f] * _B + q * _BQ
                bbase = ibg[f] * _B + q * _BQ
                k0 = kv0[f]
                k1 = kv1[f]
                k2 = kv2[f]
                k3 = kv3[f]

                @plsc.parallel_loop(0, _BQ, step=_LANE, unroll=_UNR)
                def bat(off, f=f, abase=abase, bbase=bbase, q=q,
                        k0=k0, k1=k1, k2=k2, k3=k3):
                    av = tx_v[pl.ds(abase + off, _LANE)]
                    bv = tx_v[pl.ds(bbase + off, _LANE)]
                    os_[q % 2][f, pl.ds(off, _LANE)] = (
                        (k1 + k3 * bv) * av + (k2 * bv + k0))

            pltpu.make_async_copy(os_[o_slot], ot_hbm.at[rows],
                                  sos[o_slot]).start()

    pltpu.make_async_copy(
        o0, ot_hbm.at[pl.ds(2 * _K + base, _LANE)], so0).wait()
    pltpu.make_async_copy(
        o1, ot_hbm.at[pl.ds(3 * _K + base, _LANE)], so1).wait()


_part_out = jax.ShapeDtypeStruct((_NTILES, _B), jnp.float32)


@functools.partial(
    pl.kernel,
    out_type=_part_out,
    mesh=_mesh,
    scratch_types=[
        pltpu.VMEM((_NQ, 2 * _FPT), jnp.int32),
        pltpu.VMEM((16, _FPT), jnp.float32),
        pltpu.VMEM((4 * _FPT,), jnp.float32),
        pltpu.VMEM((2 * _CH, _BQ), jnp.float32),
        pltpu.VMEM((2 * _CH, _BQ), jnp.float32),
        pltpu.VMEM((2 * _CH, _BQ), jnp.float32),
        pltpu.VMEM((2 * _CH, _BQ), jnp.float32),
        pltpu.VMEM((1, _B), jnp.float32),
        pltpu.SemaphoreType.DMA,
        pltpu.SemaphoreType.DMA,
        pltpu.SemaphoreType.DMA,
        pltpu.SemaphoreType.DMA,
    ],
    name="logic_layer_out",
)
def _layer_last(ht, wt_hbm, iab_hbm, p_hbm,
                iq_v, wt_v, kc_v, rab0, rab1, rab2, rab3, acc,
                sg0, sg1, sg2, sg3):
    wid = lax.axis_index("s") * 2 + lax.axis_index("c")
    base = wid * _FPT
    _load_quarter_indices(iab_hbm, iq_v, base)
    pltpu.sync_copy(wt_hbm.at[:, pl.ds(base, _FPT)], wt_v)
    _fold_coeffs(wt_v, kc_v)

    rabs = [rab0, rab1, rab2, rab3]
    sgs = [sg0, sg1, sg2, sg3]

    zv = wt_v[0, pl.ds(0, _LANE)] * 0.0

    @pl.loop(0, _B // _LANE)
    def zinit(i):
        acc[0, pl.ds(i * _LANE, _LANE)] = zv

    def accum_chunk(kv, lane_off, rab, q):
        kv0, kv1, kv2, kv3 = kv
        ks = [(kv0[lane_off + f], kv1[lane_off + f],
               kv2[lane_off + f], kv3[lane_off + f]) for f in range(_CH)]

        @plsc.parallel_loop(0, _BQ, step=_LANE, unroll=4)
        def bat(off):
            isl = pl.ds(off, _LANE)
            osl = pl.ds(q * _BQ + off, _LANE)
            accv = acc[0, osl]
            for f in range(_CH):
                k0, k1, k2, k3 = ks[f]
                av = rab[f, isl]
                bv = rab[_CH + f, isl]
                accv = accv + ((k1 + k3 * bv) * av + (k2 * bv + k0))
            acc[0, osl] = accv

    def gath(slot, c, q):
        idx = iq_v.at[q, pl.ds(c * 2 * _CH, 2 * _CH)]
        return pltpu.make_async_copy(ht.at[idx], rabs[slot], sgs[slot])

    for q in range(3):
        gath(q, 0, q).start()

    @pl.loop(0, _NCH // 2)
    def body(t):
        c0 = 2 * t
        c1 = 2 * t + 1
        kv = [kc_v[pl.ds(n * _FPT + t * _LANE, _LANE)] for n in range(4)]

        for u in range(8):
            c = c0 if u < 4 else c1
            lane_off = 0 if u < 4 else _CH
            q = u % 4
            slot = u % 4

            if u + 3 < 8:
                gath((u + 3) % 4, c0 if u + 3 < 4 else c1, (u + 3) % 4
                     ).start()
            else:
                @pl.when(t < _NCH // 2 - 1)
                def _(u=u):
                    gath((u + 3) % 4, c0 + 2, (u + 3) % 4).start()

            gath(slot, c, q).wait()
            accum_chunk(kv, lane_off, rabs[slot], q)

    pltpu.sync_copy(acc, p_hbm.at[pl.ds(wid, 1)])


def _combine(p):
    def body(p_ref, o_ref):
        pv = p_ref[...]
        o_ref[0, :] = jnp.sum(pv[:16, :], axis=0)
        o_ref[1, :] = jnp.sum(pv[16:, :], axis=0)

    return pl.pallas_call(
        body,
        out_shape=jax.ShapeDtypeStruct((2, _B), jnp.float32),
    )(p)


def _iab(a, b):
    return jnp.stack([a.reshape(-1, _CH), b.reshape(-1, _CH)],
                     axis=1).reshape(-1)


def kernel(x, w0, w1, w2, w3, a0, b0, a1, b1, a2, b2, a3, b3):
    x_flat = jnp.transpose(x).reshape(-1)
    h = _layer_in(x_flat, jnp.transpose(w0), a0, b0)
    h = _layer_mid(h, jnp.transpose(w1), _iab(a1, b1))
    h = _layer_mid(h, jnp.transpose(w2), _iab(a2, b2))
    p = _layer_last(h, jnp.transpose(w3), _iab(a3, b3))
    return jnp.transpose(_combine(p))
